# baseline (speedup 1.0000x reference)
import functools

import jax
import jax.numpy as jnp
from jax import lax
from jax.experimental import pallas as pl
from jax.experimental.pallas import tpu as pltpu
from jax.experimental.pallas import tpu_sc as plsc

N, E, D = 10000, 160000, 256

NC, NS, LANES = 2, 16, 16
NW = NC * NS

EP = 163840
EDGES_PER_W = EP // NW
CHUNK = 128
NCHUNK = EDGES_PER_W // CHUNK
NPAD = 10112
ROWS_PER_TILE = NPAD // NS
DC = 128
SLAB = 128
ZROWS = 40
GCH0 = 40
GCH1 = (EP // CHUNK - NS * GCH0) // NS
GMAX = max(GCH0, GCH1)


def _mesh():
    return plsc.VectorSubcoreMesh(
        core_axis_name="c", subcore_axis_name="s", num_cores=NC, num_subcores=NS)


def _node_linear_body(h_ref, w_ref, b_ref, ah, bh, dh, eh):
    y = jnp.dot(h_ref[...], w_ref[...],
                preferred_element_type=jnp.float32) + b_ref[...]
    ah[...] = y[:, 0:D]
    b = lax.bitcast_convert_type(y, jnp.uint32)
    yb = b + jnp.uint32(0x7FFF) + ((b >> jnp.uint32(16)) & jnp.uint32(1))
    for tbl, c0 in ((bh, D), (dh, 2 * D), (eh, 3 * D)):
        lo = yb[:, c0:c0 + 128] >> jnp.uint32(16)
        hi = yb[:, c0 + 128:c0 + 256] & jnp.uint32(0xFFFF0000)
        tbl[...] = lax.bitcast_convert_type(hi | lo, jnp.int32)


def _unpack_bf16pair(ref):
    w = lax.bitcast_convert_type(ref[...], jnp.uint32)
    lo = lax.bitcast_convert_type(w << jnp.uint32(16), jnp.float32)
    hi = lax.bitcast_convert_type(w & jnp.uint32(0xFFFF0000), jnp.float32)
    return jnp.concatenate([lo, hi], axis=1)


def _pack_bf16pair(x):
    b = lax.bitcast_convert_type(x, jnp.uint32)
    yb = b + jnp.uint32(0x7FFF) + ((b >> jnp.uint32(16)) & jnp.uint32(1))
    lo = yb[:, 0:128] >> jnp.uint32(16)
    hi = yb[:, 128:256] & jnp.uint32(0xFFFF0000)
    return lax.bitcast_convert_type(hi | lo, jnp.int32)


def _edge_body(e_ref, d_ref, t_ref, b_ref, cw_ref, cb_ref,
               eij, ml, mh, sl, sh, stats):
    i = pl.program_id(0)
    x = jnp.dot(e_ref[...], cw_ref[...], preferred_element_type=jnp.float32)
    x = x + cb_ref[...] + _unpack_bf16pair(d_ref) + _unpack_bf16pair(t_ref)
    sg = jax.nn.sigmoid(x)
    eij[...] = _pack_bf16pair(x)
    msg = sg * _unpack_bf16pair(b_ref)
    ml[...] = msg[:, 0:128]
    mh[...] = msg[:, 128:256]
    sl[...] = sg[:, 0:128]
    sh[...] = sg[:, 128:256]

    @pl.when(i == 0)
    def _():
        stats[...] = jnp.zeros_like(stats)

    stats[0:1, :] += jnp.sum(x, axis=0, keepdims=True)
    stats[1:2, :] += jnp.sum(x * x, axis=0, keepdims=True)


def _agg_body(ah_ref, num_ref, den_ref, hagg, stats):
    i = pl.program_id(0)
    nm = num_ref[0] + num_ref[1]
    dn = den_ref[0] + den_ref[1]
    x = ah_ref[...] + nm / (dn + 1e-6)
    hagg[...] = x

    @pl.when(i == 0)
    def _():
        stats[...] = jnp.zeros_like(stats)

    stats[0:1, :] += jnp.sum(x, axis=0, keepdims=True)
    stats[1:2, :] += jnp.sum(x * x, axis=0, keepdims=True)


def _bn_residual_body(nrows, packed, res_ref, x_ref, stats_ref, g_ref, b_ref, out):
    st = stats_ref[...]
    mean = st[0:1, :] / nrows
    var = st[1:2, :] / nrows - mean * mean
    inv = lax.rsqrt(var + 1e-5)
    x = _unpack_bf16pair(x_ref) if packed else x_ref[...]
    xn = (x - mean) * inv * g_ref[...] + b_ref[...]
    out[...] = res_ref[...] + jnp.maximum(xn, 0.0)


@functools.cache
def _gather_sc():
    return pl.kernel(
        _gather_sc_body,
        out_type=(jax.ShapeDtypeStruct((EP, 128), jnp.int32),) * 3,
        mesh=_mesh(),
        scratch_types=[
            pltpu.VMEM((GMAX, CHUNK), jnp.int32),
            pltpu.VMEM((GMAX, CHUNK), jnp.int32),
            pltpu.VMEM((CHUNK, 128), jnp.int32),
            pltpu.VMEM((CHUNK, 128), jnp.int32),
            pltpu.VMEM((CHUNK, 128), jnp.int32),
            pltpu.SemaphoreType.DMA,
            pltpu.SemaphoreType.DMA,
            pltpu.SemaphoreType.DMA,
        ],
    )


def _gather_sc_body(dh, eh, bh, src2d, dst2d, dsrc, edst, bsrc,
                    sidx, didx, dbuf, ebuf, bbuf, s0, s1, s2):
    c = lax.axis_index("c")
    s = lax.axis_index("s")
    nch = jnp.where(c == 0, GCH0, GCH1)
    row0 = jnp.where(c == 0, s * GCH0, NS * GCH0 + s * GCH1)
    pltpu.sync_copy(src2d.at[pl.ds(row0, GMAX)], sidx)
    pltpu.sync_copy(dst2d.at[pl.ds(row0, GMAX)], didx)

    def body(j, carry):
        base = (row0 + j) * CHUNK
        cp0 = pltpu.async_copy(dh.at[sidx.at[j]], dbuf, s0)
        cp1 = pltpu.async_copy(eh.at[didx.at[j]], ebuf, s1)
        cp2 = pltpu.async_copy(bh.at[sidx.at[j]], bbuf, s2)
        cp0.wait()
        cp1.wait()
        cp2.wait()
        pltpu.sync_copy(dbuf, dsrc.at[pl.ds(base, CHUNK)])
        pltpu.sync_copy(ebuf, edst.at[pl.ds(base, CHUNK)])
        pltpu.sync_copy(bbuf, bsrc.at[pl.ds(base, CHUNK)])
        return carry

    lax.fori_loop(0, nch, body, 0)


@functools.cache
def _scatter_sc():
    return pl.kernel(
        _scatter_sc_body,
        out_type=(jax.ShapeDtypeStruct((NC, NPAD, D), jnp.float32),) * 2,
        mesh=_mesh(),
        scratch_types=[
            pltpu.VMEM((NCHUNK, CHUNK), jnp.int32),
            pltpu.VMEM((2, SLAB, DC), jnp.float32),
            pltpu.VMEM((ZROWS, DC), jnp.float32),
            pltpu.VMEM_SHARED((NPAD, DC), jnp.float32),
            pltpu.SemaphoreType.DMA,
            pltpu.SemaphoreType.DMA,
            pltpu.SemaphoreType.DMA,
            pltpu.SemaphoreType.DMA,
        ],
    )


def _scatter_sc_body(ml, mh, sl, sh, dst2d, num2, den2,
                     didx, buf, zbuf, acc, l0, l1, s0, s1):
    c = lax.axis_index("c")
    s = lax.axis_index("s")
    w = c * NS + s
    row0 = w * NCHUNK
    pltpu.sync_copy(dst2d.at[pl.ds(row0, NCHUNK)], didx)

    def zbody(i, carry):
        r = i // (DC // LANES)
        q = i % (DC // LANES)
        zbuf[r, pl.ds(q * LANES, LANES)] = jnp.zeros((LANES,), jnp.float32)
        return carry

    lax.fori_loop(0, ZROWS * (DC // LANES), zbody, 0)
    tr0 = s * ROWS_PER_TILE

    for p, arr in enumerate((ml, mh, sl, sh)):
        outp = num2 if p < 2 else den2
        hcol = (p % 2) * DC
        for z in range(ROWS_PER_TILE // ZROWS):
            pltpu.sync_copy(zbuf, acc.at[pl.ds(tr0 + z * ZROWS, ZROWS)])
        zrem = ROWS_PER_TILE % ZROWS
        if zrem:
            pltpu.sync_copy(
                zbuf.at[pl.ds(0, zrem)],
                acc.at[pl.ds(tr0 + (ROWS_PER_TILE // ZROWS) * ZROWS, zrem)])
        plsc.subcore_barrier()

        def issue_l(j, slot, lsem):
            ebase = w * EDGES_PER_W + j * SLAB
            pltpu.async_copy(arr.at[pl.ds(ebase, SLAB)], buf.at[slot], lsem)

        def wait_l(slot, lsem):
            pltpu.make_async_copy(arr.at[pl.ds(0, SLAB)], buf.at[slot],
                                  lsem).wait()

        def issue_s(j, slot, ssem):
            pltpu.async_copy(buf.at[slot], acc.at[didx.at[j]], ssem, add=True)

        def wait_s(slot, ssem):
            pltpu.make_async_copy(buf.at[slot], acc.at[didx.at[0]],
                                  ssem).wait()

        issue_l(0, 0, l0)

        def sbody(t, carry):
            j0 = 2 * t
            j1 = 2 * t + 1
            wait_l(0, l0)

            @pl.when(t > 0)
            def _():
                wait_s(1, s1)

            issue_l(j1, 1, l1)
            issue_s(j0, 0, s0)
            wait_l(1, l1)
            wait_s(0, s0)

            @pl.when(t < EDGES_PER_W // SLAB // 2 - 1)
            def _():
                issue_l(j0 + 2, 0, l0)

            issue_s(j1, 1, s1)
            return carry

        lax.fori_loop(0, EDGES_PER_W // SLAB // 2, sbody, 0)
        wait_s(1, s1)
        plsc.subcore_barrier()
        pltpu.sync_copy(acc.at[pl.ds(tr0, ROWS_PER_TILE)],
                        outp.at[c, pl.ds(tr0, ROWS_PER_TILE), pl.ds(hcol, DC)])
        plsc.subcore_barrier()


BN1 = 2000
BE = 3200


def kernel(h, e, edge_index, A_w, A_b, B_w, B_b, C_w, C_b, D_w, D_b, E_w, E_b,
           bn_h_gamma, bn_h_beta, bn_e_gamma, bn_e_beta):
    f32 = jnp.float32
    src = edge_index[0].astype(jnp.int32)
    dst = edge_index[1].astype(jnp.int32)
    pad = EP - E
    src2d = jnp.concatenate([src, jnp.zeros((pad,), jnp.int32)]).reshape(EP // CHUNK, CHUNK)
    dst2d_g = jnp.concatenate([dst, jnp.zeros((pad,), jnp.int32)]).reshape(EP // CHUNK, CHUNK)
    dst2d_s = jnp.concatenate([dst, jnp.full((pad,), N, jnp.int32)]).reshape(EP // CHUNK, CHUNK)

    Wc = jnp.concatenate([A_w, B_w, D_w, E_w], axis=0).T
    bc = jnp.concatenate([A_b, B_b, D_b, E_b]).reshape(1, 4 * D)
    CwT = C_w.T
    cb = C_b.reshape(1, D)

    ah, bh, dh, eh = pl.pallas_call(
        _node_linear_body,
        grid=(N // BN1,),
        in_specs=[pl.BlockSpec((BN1, D), lambda i: (i, 0)),
                  pl.BlockSpec((D, 4 * D), lambda i: (0, 0)),
                  pl.BlockSpec((1, 4 * D), lambda i: (0, 0))],
        out_specs=[pl.BlockSpec((BN1, D), lambda i: (i, 0))] + [
            pl.BlockSpec((BN1, 128), lambda i: (i, 0))] * 3,
        out_shape=[jax.ShapeDtypeStruct((N, D), f32)] + [
            jax.ShapeDtypeStruct((N, 128), jnp.int32)] * 3,
    )(h, Wc, bc)

    dsrc, edst, bsrc = _gather_sc()(dh, eh, bh, src2d, dst2d_g)

    eij, ml, mh, sl, sh, estats = pl.pallas_call(
        _edge_body,
        grid=(E // BE,),
        in_specs=[pl.BlockSpec((BE, D), lambda i: (i, 0))] + [
            pl.BlockSpec((BE, 128), lambda i: (i, 0))] * 3 + [
            pl.BlockSpec((D, D), lambda i: (0, 0)),
            pl.BlockSpec((1, D), lambda i: (0, 0))],
        out_specs=[pl.BlockSpec((BE, 128), lambda i: (i, 0))] * 5 + [
            pl.BlockSpec((2, D), lambda i: (0, 0))],
        out_shape=[jax.ShapeDtypeStruct((E, 128), jnp.int32)] + [
            jax.ShapeDtypeStruct((EP, 128), f32)] * 4 + [
            jax.ShapeDtypeStruct((2, D), f32)],
    )(e, dsrc, edst, bsrc, CwT, cb)

    num2, den2 = _scatter_sc()(ml, mh, sl, sh, dst2d_s)

    hagg, hstats = pl.pallas_call(
        _agg_body,
        grid=(N // BN1,),
        in_specs=[pl.BlockSpec((BN1, D), lambda i: (i, 0)),
                  pl.BlockSpec((NC, BN1, D), lambda i: (0, i, 0)),
                  pl.BlockSpec((NC, BN1, D), lambda i: (0, i, 0))],
        out_specs=[pl.BlockSpec((BN1, D), lambda i: (i, 0)),
                   pl.BlockSpec((2, D), lambda i: (0, 0))],
        out_shape=[jax.ShapeDtypeStruct((N, D), f32),
                   jax.ShapeDtypeStruct((2, D), f32)],
    )(ah, num2, den2)

    h_out = pl.pallas_call(
        functools.partial(_bn_residual_body, float(N), False),
        grid=(N // BN1,),
        in_specs=[pl.BlockSpec((BN1, D), lambda i: (i, 0)),
                  pl.BlockSpec((BN1, D), lambda i: (i, 0)),
                  pl.BlockSpec((2, D), lambda i: (0, 0)),
                  pl.BlockSpec((1, D), lambda i: (0, 0)),
                  pl.BlockSpec((1, D), lambda i: (0, 0))],
        out_specs=pl.BlockSpec((BN1, D), lambda i: (i, 0)),
        out_shape=jax.ShapeDtypeStruct((N, D), f32),
    )(h, hagg, hstats, bn_h_gamma.reshape(1, D), bn_h_beta.reshape(1, D))

    e_out = pl.pallas_call(
        functools.partial(_bn_residual_body, float(E), True),
        grid=(E // BE,),
        in_specs=[pl.BlockSpec((BE, D), lambda i: (i, 0)),
                  pl.BlockSpec((BE, 128), lambda i: (i, 0)),
                  pl.BlockSpec((2, D), lambda i: (0, 0)),
                  pl.BlockSpec((1, D), lambda i: (0, 0)),
                  pl.BlockSpec((1, D), lambda i: (0, 0))],
        out_specs=pl.BlockSpec((BE, D), lambda i: (i, 0)),
        out_shape=jax.ShapeDtypeStruct((E, D), f32),
    )(e, eij, estats, bn_e_gamma.reshape(1, D), bn_e_beta.reshape(1, D))

    return (h_out, e_out)


# speedup vs baseline: 1.0009x; 1.0009x over previous
import functools

import jax
import jax.numpy as jnp
from jax import lax
from jax.experimental import pallas as pl
from jax.experimental.pallas import tpu as pltpu
from jax.experimental.pallas import tpu_sc as plsc

N, E, D = 10000, 160000, 256

NC, NS, LANES = 2, 16, 16
NW = NC * NS

EP = 163840
EDGES_PER_W = EP // NW
CHUNK = 128
NCHUNK = EDGES_PER_W // CHUNK
NPAD = 10112
ROWS_PER_TILE = NPAD // NS
DC = 128
SLAB = 128
ZROWS = 40
GCH0 = 40
GCH1 = (EP // CHUNK - NS * GCH0) // NS
GMAX = max(GCH0, GCH1)


def _mesh():
    return plsc.VectorSubcoreMesh(
        core_axis_name="c", subcore_axis_name="s", num_cores=NC, num_subcores=NS)


def _node_linear_body(h_ref, w_ref, b_ref, ah, bh, dh, eh):
    y = jnp.dot(h_ref[...], w_ref[...],
                preferred_element_type=jnp.float32) + b_ref[...]
    ah[...] = y[:, 0:D]
    b = lax.bitcast_convert_type(y, jnp.uint32)
    yb = b + jnp.uint32(0x7FFF) + ((b >> jnp.uint32(16)) & jnp.uint32(1))
    for tbl, c0 in ((bh, D), (dh, 2 * D), (eh, 3 * D)):
        lo = yb[:, c0:c0 + 128] >> jnp.uint32(16)
        hi = yb[:, c0 + 128:c0 + 256] & jnp.uint32(0xFFFF0000)
        tbl[...] = lax.bitcast_convert_type(hi | lo, jnp.int32)


def _unpack_bf16pair(ref):
    w = lax.bitcast_convert_type(ref[...], jnp.uint32)
    lo = lax.bitcast_convert_type(w << jnp.uint32(16), jnp.float32)
    hi = lax.bitcast_convert_type(w & jnp.uint32(0xFFFF0000), jnp.float32)
    return jnp.concatenate([lo, hi], axis=1)


def _pack_bf16pair(x):
    b = lax.bitcast_convert_type(x, jnp.uint32)
    yb = b + jnp.uint32(0x7FFF) + ((b >> jnp.uint32(16)) & jnp.uint32(1))
    lo = yb[:, 0:128] >> jnp.uint32(16)
    hi = yb[:, 128:256] & jnp.uint32(0xFFFF0000)
    return lax.bitcast_convert_type(hi | lo, jnp.int32)


def _edge_body(e_ref, d_ref, t_ref, b_ref, cw_ref, cb_ref,
               eij, ml, mh, sl, sh, stats):
    i = pl.program_id(0)
    x = jnp.dot(e_ref[...], cw_ref[...], preferred_element_type=jnp.float32)
    x = x + cb_ref[...] + _unpack_bf16pair(d_ref) + _unpack_bf16pair(t_ref)
    sg = jax.nn.sigmoid(x)
    eij[...] = _pack_bf16pair(x)
    msg = sg * _unpack_bf16pair(b_ref)
    ml[...] = msg[:, 0:128]
    mh[...] = msg[:, 128:256]
    sl[...] = sg[:, 0:128]
    sh[...] = sg[:, 128:256]

    @pl.when(i == 0)
    def _():
        stats[...] = jnp.zeros_like(stats)

    stats[0:1, :] += jnp.sum(x, axis=0, keepdims=True)
    stats[1:2, :] += jnp.sum(x * x, axis=0, keepdims=True)


def _agg_body(ah_ref, num_ref, den_ref, hagg, stats):
    i = pl.program_id(0)
    nm = num_ref[0] + num_ref[1]
    dn = den_ref[0] + den_ref[1]
    x = ah_ref[...] + nm / (dn + 1e-6)
    hagg[...] = x

    @pl.when(i == 0)
    def _():
        stats[...] = jnp.zeros_like(stats)

    stats[0:1, :] += jnp.sum(x, axis=0, keepdims=True)
    stats[1:2, :] += jnp.sum(x * x, axis=0, keepdims=True)


def _bn_residual_body(nrows, packed, res_ref, x_ref, stats_ref, g_ref, b_ref, out):
    st = stats_ref[...]
    mean = st[0:1, :] / nrows
    var = st[1:2, :] / nrows - mean * mean
    inv = lax.rsqrt(var + 1e-5)
    x = _unpack_bf16pair(x_ref) if packed else x_ref[...]
    xn = (x - mean) * inv * g_ref[...] + b_ref[...]
    out[...] = res_ref[...] + jnp.maximum(xn, 0.0)


@functools.cache
def _gather_sc():
    return pl.kernel(
        _gather_sc_body,
        out_type=(jax.ShapeDtypeStruct((EP, 128), jnp.int32),) * 3,
        mesh=_mesh(),
        scratch_types=[
            pltpu.VMEM((GMAX, CHUNK), jnp.int32),
            pltpu.VMEM((GMAX, CHUNK), jnp.int32),
            pltpu.VMEM((CHUNK, 128), jnp.int32),
            pltpu.VMEM((CHUNK, 128), jnp.int32),
            pltpu.VMEM((CHUNK, 128), jnp.int32),
            pltpu.SemaphoreType.DMA,
            pltpu.SemaphoreType.DMA,
            pltpu.SemaphoreType.DMA,
        ],
    )


def _gather_sc_body(dh, eh, bh, src2d, dst2d, dsrc, edst, bsrc,
                    sidx, didx, dbuf, ebuf, bbuf, s0, s1, s2):
    c = lax.axis_index("c")
    s = lax.axis_index("s")
    nch = jnp.where(c == 0, GCH0, GCH1)
    row0 = jnp.where(c == 0, s * GCH0, NS * GCH0 + s * GCH1)
    pltpu.sync_copy(src2d.at[pl.ds(row0, GMAX)], sidx)
    pltpu.sync_copy(dst2d.at[pl.ds(row0, GMAX)], didx)

    def body(j, carry):
        base = (row0 + j) * CHUNK
        cp0 = pltpu.async_copy(dh.at[sidx.at[j]], dbuf, s0)
        cp1 = pltpu.async_copy(eh.at[didx.at[j]], ebuf, s1)
        cp2 = pltpu.async_copy(bh.at[sidx.at[j]], bbuf, s2)
        cp0.wait()
        cp1.wait()
        cp2.wait()
        pltpu.sync_copy(dbuf, dsrc.at[pl.ds(base, CHUNK)])
        pltpu.sync_copy(ebuf, edst.at[pl.ds(base, CHUNK)])
        pltpu.sync_copy(bbuf, bsrc.at[pl.ds(base, CHUNK)])
        return carry

    lax.fori_loop(0, nch, body, 0)


@functools.cache
def _scatter_sc():
    return pl.kernel(
        _scatter_sc_body,
        out_type=(jax.ShapeDtypeStruct((NC, NPAD, D), jnp.float32),) * 2,
        mesh=_mesh(),
        scratch_types=[
            pltpu.VMEM((NCHUNK, CHUNK), jnp.int32),
            pltpu.VMEM((2, SLAB, DC), jnp.float32),
            pltpu.VMEM((ZROWS, DC), jnp.float32),
            pltpu.VMEM_SHARED((NPAD, DC), jnp.float32),
            pltpu.SemaphoreType.DMA,
            pltpu.SemaphoreType.DMA,
            pltpu.SemaphoreType.DMA,
            pltpu.SemaphoreType.DMA,
        ],
    )


def _scatter_sc_body(ml, mh, sl, sh, dst2d, num2, den2,
                     didx, buf, zbuf, acc, l0, l1, s0, s1):
    c = lax.axis_index("c")
    s = lax.axis_index("s")
    w = c * NS + s
    row0 = w * NCHUNK
    pltpu.sync_copy(dst2d.at[pl.ds(row0, NCHUNK)], didx)

    def zbody(i, carry):
        r = i // (DC // LANES)
        q = i % (DC // LANES)
        zbuf[r, pl.ds(q * LANES, LANES)] = jnp.zeros((LANES,), jnp.float32)
        return carry

    lax.fori_loop(0, ZROWS * (DC // LANES), zbody, 0)
    tr0 = s * ROWS_PER_TILE

    for p, arr in enumerate((ml, mh, sl, sh)):
        outp = num2 if p < 2 else den2
        hcol = (p % 2) * DC
        for z in range(ROWS_PER_TILE // ZROWS):
            pltpu.sync_copy(zbuf, acc.at[pl.ds(tr0 + z * ZROWS, ZROWS)])
        zrem = ROWS_PER_TILE % ZROWS
        if zrem:
            pltpu.sync_copy(
                zbuf.at[pl.ds(0, zrem)],
                acc.at[pl.ds(tr0 + (ROWS_PER_TILE // ZROWS) * ZROWS, zrem)])
        plsc.subcore_barrier()

        def issue_l(j, slot, lsem):
            ebase = w * EDGES_PER_W + j * SLAB
            pltpu.async_copy(arr.at[pl.ds(ebase, SLAB)], buf.at[slot], lsem)

        def wait_l(slot, lsem):
            pltpu.make_async_copy(arr.at[pl.ds(0, SLAB)], buf.at[slot],
                                  lsem).wait()

        def issue_s(j, slot, ssem):
            pltpu.async_copy(buf.at[slot], acc.at[didx.at[j]], ssem, add=True)

        def wait_s(slot, ssem):
            pltpu.make_async_copy(buf.at[slot], acc.at[didx.at[0]],
                                  ssem).wait()

        issue_l(0, 0, l0)

        def sbody(t, carry):
            j0 = 2 * t
            j1 = 2 * t + 1
            wait_l(0, l0)

            @pl.when(t > 0)
            def _():
                wait_s(1, s1)

            issue_l(j1, 1, l1)
            issue_s(j0, 0, s0)
            wait_l(1, l1)
            wait_s(0, s0)

            @pl.when(t < EDGES_PER_W // SLAB // 2 - 1)
            def _():
                issue_l(j0 + 2, 0, l0)

            issue_s(j1, 1, s1)
            return carry

        lax.fori_loop(0, EDGES_PER_W // SLAB // 2, sbody, 0)
        wait_s(1, s1)
        plsc.subcore_barrier()
        pltpu.sync_copy(acc.at[pl.ds(tr0, ROWS_PER_TILE)],
                        outp.at[c, pl.ds(tr0, ROWS_PER_TILE), pl.ds(hcol, DC)])
        plsc.subcore_barrier()


BN1 = 2000
BE = 2000


def kernel(h, e, edge_index, A_w, A_b, B_w, B_b, C_w, C_b, D_w, D_b, E_w, E_b,
           bn_h_gamma, bn_h_beta, bn_e_gamma, bn_e_beta):
    f32 = jnp.float32
    src = edge_index[0].astype(jnp.int32)
    dst = edge_index[1].astype(jnp.int32)
    pad = EP - E
    src2d = jnp.concatenate([src, jnp.zeros((pad,), jnp.int32)]).reshape(EP // CHUNK, CHUNK)
    dst2d_g = jnp.concatenate([dst, jnp.zeros((pad,), jnp.int32)]).reshape(EP // CHUNK, CHUNK)
    dst2d_s = jnp.concatenate([dst, jnp.full((pad,), N, jnp.int32)]).reshape(EP // CHUNK, CHUNK)

    Wc = jnp.concatenate([A_w, B_w, D_w, E_w], axis=0).T
    bc = jnp.concatenate([A_b, B_b, D_b, E_b]).reshape(1, 4 * D)
    CwT = C_w.T
    cb = C_b.reshape(1, D)

    ah, bh, dh, eh = pl.pallas_call(
        _node_linear_body,
        grid=(N // BN1,),
        in_specs=[pl.BlockSpec((BN1, D), lambda i: (i, 0)),
                  pl.BlockSpec((D, 4 * D), lambda i: (0, 0)),
                  pl.BlockSpec((1, 4 * D), lambda i: (0, 0))],
        out_specs=[pl.BlockSpec((BN1, D), lambda i: (i, 0))] + [
            pl.BlockSpec((BN1, 128), lambda i: (i, 0))] * 3,
        out_shape=[jax.ShapeDtypeStruct((N, D), f32)] + [
            jax.ShapeDtypeStruct((N, 128), jnp.int32)] * 3,
    )(h, Wc, bc)

    dsrc, edst, bsrc = _gather_sc()(dh, eh, bh, src2d, dst2d_g)

    eij, ml, mh, sl, sh, estats = pl.pallas_call(
        _edge_body,
        grid=(E // BE,),
        in_specs=[pl.BlockSpec((BE, D), lambda i: (i, 0))] + [
            pl.BlockSpec((BE, 128), lambda i: (i, 0))] * 3 + [
            pl.BlockSpec((D, D), lambda i: (0, 0)),
            pl.BlockSpec((1, D), lambda i: (0, 0))],
        out_specs=[pl.BlockSpec((BE, 128), lambda i: (i, 0))] * 5 + [
            pl.BlockSpec((2, D), lambda i: (0, 0))],
        out_shape=[jax.ShapeDtypeStruct((E, 128), jnp.int32)] + [
            jax.ShapeDtypeStruct((EP, 128), f32)] * 4 + [
            jax.ShapeDtypeStruct((2, D), f32)],
    )(e, dsrc, edst, bsrc, CwT, cb)

    num2, den2 = _scatter_sc()(ml, mh, sl, sh, dst2d_s)

    hagg, hstats = pl.pallas_call(
        _agg_body,
        grid=(N // BN1,),
        in_specs=[pl.BlockSpec((BN1, D), lambda i: (i, 0)),
                  pl.BlockSpec((NC, BN1, D), lambda i: (0, i, 0)),
                  pl.BlockSpec((NC, BN1, D), lambda i: (0, i, 0))],
        out_specs=[pl.BlockSpec((BN1, D), lambda i: (i, 0)),
                   pl.BlockSpec((2, D), lambda i: (0, 0))],
        out_shape=[jax.ShapeDtypeStruct((N, D), f32),
                   jax.ShapeDtypeStruct((2, D), f32)],
    )(ah, num2, den2)

    h_out = pl.pallas_call(
        functools.partial(_bn_residual_body, float(N), False),
        grid=(N // BN1,),
        in_specs=[pl.BlockSpec((BN1, D), lambda i: (i, 0)),
                  pl.BlockSpec((BN1, D), lambda i: (i, 0)),
                  pl.BlockSpec((2, D), lambda i: (0, 0)),
                  pl.BlockSpec((1, D), lambda i: (0, 0)),
                  pl.BlockSpec((1, D), lambda i: (0, 0))],
        out_specs=pl.BlockSpec((BN1, D), lambda i: (i, 0)),
        out_shape=jax.ShapeDtypeStruct((N, D), f32),
    )(h, hagg, hstats, bn_h_gamma.reshape(1, D), bn_h_beta.reshape(1, D))

    e_out = pl.pallas_call(
        functools.partial(_bn_residual_body, float(E), True),
        grid=(E // BE,),
        in_specs=[pl.BlockSpec((BE, D), lambda i: (i, 0)),
                  pl.BlockSpec((BE, 128), lambda i: (i, 0)),
                  pl.BlockSpec((2, D), lambda i: (0, 0)),
                  pl.BlockSpec((1, D), lambda i: (0, 0)),
                  pl.BlockSpec((1, D), lambda i: (0, 0))],
        out_specs=pl.BlockSpec((BE, D), lambda i: (i, 0)),
        out_shape=jax.ShapeDtypeStruct((E, D), f32),
    )(e, eij, estats, bn_e_gamma.reshape(1, D), bn_e_beta.reshape(1, D))

    return (h_out, e_out)
